# Initial kernel scaffold; baseline (speedup 1.0000x reference)
import jax
import jax.numpy as jnp
from jax.experimental import pallas as pl


def kernel(site_energy, segment_ids, num_crystals):
    raise NotImplementedError("write your pallas kernel here")


# speedup vs baseline: 22.9278x; 22.9278x over previous
import functools

import jax
import jax.numpy as jnp
from jax import lax
from jax.experimental import pallas as pl
from jax.experimental.pallas import tpu as pltpu
from jax.experimental.pallas import tpu_sc as plsc

_NUM_SEGMENTS = 100000
_LANE = 128
_NC = 2
_NS = 16
_NW = _NC * _NS
_KROWS = 8

_SP = ((_NUM_SEGMENTS + _NS * _LANE - 1) // (_NS * _LANE)) * (_NS * _LANE)
_STRIPE = _SP // _NS


def _make_sc_accumulate(rows_per_worker, nchunks):
    mesh = plsc.VectorSubcoreMesh(core_axis_name="c", subcore_axis_name="s")

    @functools.partial(
        pl.kernel,
        mesh=mesh,
        out_type=(
            jax.ShapeDtypeStruct((_NC, _SP), jnp.float32),
            jax.ShapeDtypeStruct((_NC, _SP), jnp.float32),
        ),
        scratch_types=[
            pltpu.VMEM((_KROWS, _LANE), jnp.float32),
            pltpu.VMEM((_KROWS, _LANE), jnp.int32),
            pltpu.VMEM((_LANE,), jnp.float32),
            pltpu.VMEM((_STRIPE,), jnp.float32),
            pltpu.VMEM_SHARED((_SP,), jnp.float32),
            pltpu.VMEM_SHARED((_SP,), jnp.float32),
        ],
    )
    def sc_k(e_hbm, id_hbm, psum_hbm, pcnt_hbm,
             ebuf, idbuf, ones, zbuf, sums_sh, cnts_sh):
        c = lax.axis_index("c")
        s = lax.axis_index("s")
        wid = s * _NC + c

        zero16 = jnp.zeros((16,), jnp.float32)
        for i in range(_LANE // 16):
            ones[pl.ds(i * 16, 16)] = zero16 + 1.0

        def zfill(i, carry):
            zbuf[pl.ds(i * 16, 16)] = zero16
            return carry

        lax.fori_loop(0, _STRIPE // 16, zfill, 0)
        pltpu.sync_copy(zbuf, sums_sh.at[pl.ds(s * _STRIPE, _STRIPE)])
        pltpu.sync_copy(zbuf, cnts_sh.at[pl.ds(s * _STRIPE, _STRIPE)])
        plsc.subcore_barrier()

        base = wid * rows_per_worker

        def chunk(i, carry):
            r0 = base + i * _KROWS
            pltpu.sync_copy(e_hbm.at[pl.ds(r0, _KROWS)], ebuf)
            pltpu.sync_copy(id_hbm.at[pl.ds(r0, _KROWS)], idbuf)
            for j in range(_KROWS):
                pltpu.sync_copy(ebuf.at[j], sums_sh.at[idbuf.at[j]], add=True)
                pltpu.sync_copy(ones, cnts_sh.at[idbuf.at[j]], add=True)
            return carry

        lax.fori_loop(0, nchunks, chunk, 0)
        plsc.subcore_barrier()

        sl = pl.ds(s * _STRIPE, _STRIPE)
        pltpu.sync_copy(sums_sh.at[sl], psum_hbm.at[c, sl])
        pltpu.sync_copy(cnts_sh.at[sl], pcnt_hbm.at[c, sl])

    return sc_k


def _tc_finalize(ps_ref, pc_ref, o_ref):
    total = ps_ref[0] + ps_ref[1]
    count = pc_ref[0] + pc_ref[1]
    o_ref[...] = total / jnp.maximum(count, 1.0)


def kernel(site_energy, segment_ids, num_crystals):
    n = site_energy.shape[0]
    flat = site_energy.reshape(n)

    block = _NW * _KROWS * _LANE
    n_pad = ((n + block - 1) // block) * block
    pad = n_pad - n
    flat = jnp.pad(flat, (0, pad))
    ids = jnp.pad(segment_ids, (0, pad), constant_values=_NUM_SEGMENTS)

    rows = n_pad // _LANE
    rows_per_worker = rows // _NW
    nchunks = rows_per_worker // _KROWS

    e2d = flat.reshape(rows, _LANE)
    id2d = ids.reshape(rows, _LANE)

    psum, pcnt = _make_sc_accumulate(rows_per_worker, nchunks)(e2d, id2d)

    srows = _SP // _LANE
    mean2d = pl.pallas_call(
        _tc_finalize,
        out_shape=jax.ShapeDtypeStruct((srows, _LANE), jnp.float32),
    )(psum.reshape(_NC, srows, _LANE), pcnt.reshape(_NC, srows, _LANE))

    return mean2d.reshape(_SP)[:_NUM_SEGMENTS, None]


# speedup vs baseline: 32.3609x; 1.4114x over previous
import functools

import jax
import jax.numpy as jnp
from jax import lax
from jax.experimental import pallas as pl
from jax.experimental.pallas import tpu as pltpu
from jax.experimental.pallas import tpu_sc as plsc

_NUM_SEGMENTS = 100000
_LANE = 128
_NC = 2
_NS = 16
_NW = _NC * _NS
_KROWS = 8

_SP = ((_NUM_SEGMENTS + _NS * _LANE - 1) // (_NS * _LANE)) * (_NS * _LANE)
_STRIPE = _SP // _NS


def _make_sc_accumulate(rows_per_worker, nchunks):
    mesh = plsc.VectorSubcoreMesh(core_axis_name="c", subcore_axis_name="s")

    @functools.partial(
        pl.kernel,
        mesh=mesh,
        out_type=(
            jax.ShapeDtypeStruct((_NC, _SP), jnp.float32),
            jax.ShapeDtypeStruct((_NC, _SP), jnp.float32),
        ),
        scratch_types=[
            pltpu.VMEM((2, _KROWS, _LANE), jnp.float32),
            pltpu.VMEM((2, _KROWS, _LANE), jnp.int32),
            pltpu.VMEM((_LANE,), jnp.float32),
            pltpu.VMEM((_STRIPE,), jnp.float32),
            pltpu.VMEM_SHARED((_SP,), jnp.float32),
            pltpu.VMEM_SHARED((_SP,), jnp.float32),
            pltpu.SemaphoreType.DMA,
            pltpu.SemaphoreType.DMA,
            pltpu.SemaphoreType.DMA,
        ],
    )
    def sc_k(e_hbm, id_hbm, psum_hbm, pcnt_hbm,
             ebuf, idbuf, ones, zbuf, sums_sh, cnts_sh,
             sem_l0, sem_l1, sem_s):
        c = lax.axis_index("c")
        s = lax.axis_index("s")
        wid = s * _NC + c

        zero16 = jnp.zeros((16,), jnp.float32)
        for i in range(_LANE // 16):
            ones[pl.ds(i * 16, 16)] = zero16 + 1.0

        def zfill(i, carry):
            zbuf[pl.ds(i * 16, 16)] = zero16
            return carry

        lax.fori_loop(0, _STRIPE // 16, zfill, 0)
        pltpu.sync_copy(zbuf, sums_sh.at[pl.ds(s * _STRIPE, _STRIPE)])
        pltpu.sync_copy(zbuf, cnts_sh.at[pl.ds(s * _STRIPE, _STRIPE)])
        plsc.subcore_barrier()

        base = wid * rows_per_worker
        sem_l = (sem_l0, sem_l1)

        def start_loads(i, b):
            r0 = base + i * _KROWS
            pltpu.async_copy(e_hbm.at[pl.ds(r0, _KROWS)], ebuf.at[b], sem_l[b])
            pltpu.async_copy(id_hbm.at[pl.ds(r0, _KROWS)], idbuf.at[b], sem_l[b])

        def wait_loads(b):
            pltpu.make_async_copy(
                e_hbm.at[pl.ds(0, _KROWS)], ebuf.at[b], sem_l[b]).wait()
            pltpu.make_async_copy(
                id_hbm.at[pl.ds(0, _KROWS)], idbuf.at[b], sem_l[b]).wait()

        start_loads(0, 0)
        start_loads(1, 1)

        def pair(k, carry):
            for b in range(2):
                i = k * 2 + b
                wait_loads(b)
                descs = []
                for j in range(_KROWS):
                    descs.append(pltpu.async_copy(
                        ebuf.at[b, j], sums_sh.at[idbuf.at[b, j]], sem_s,
                        add=True))
                    descs.append(pltpu.async_copy(
                        ones, cnts_sh.at[idbuf.at[b, j]], sem_s, add=True))
                for d in descs:
                    d.wait()

                @pl.when(i + 2 < nchunks)
                def _():
                    start_loads(i + 2, b)
            return carry

        lax.fori_loop(0, nchunks // 2, pair, 0)
        plsc.subcore_barrier()

        sl = pl.ds(s * _STRIPE, _STRIPE)
        pltpu.sync_copy(sums_sh.at[sl], psum_hbm.at[c, sl])
        pltpu.sync_copy(cnts_sh.at[sl], pcnt_hbm.at[c, sl])

    return sc_k


def _tc_finalize(ps_ref, pc_ref, o_ref):
    total = ps_ref[0] + ps_ref[1]
    count = pc_ref[0] + pc_ref[1]
    o_ref[...] = total / jnp.maximum(count, 1.0)


def kernel(site_energy, segment_ids, num_crystals):
    n = site_energy.shape[0]
    flat = site_energy.reshape(n)

    block = _NW * _KROWS * _LANE
    n_pad = ((n + block - 1) // block) * block
    pad = n_pad - n
    flat = jnp.pad(flat, (0, pad))
    ids = jnp.pad(segment_ids, (0, pad), constant_values=_NUM_SEGMENTS)

    rows = n_pad // _LANE
    rows_per_worker = rows // _NW
    nchunks = rows_per_worker // _KROWS

    e2d = flat.reshape(rows, _LANE)
    id2d = ids.reshape(rows, _LANE)

    psum, pcnt = _make_sc_accumulate(rows_per_worker, nchunks)(e2d, id2d)

    srows = _SP // _LANE
    mean2d = pl.pallas_call(
        _tc_finalize,
        out_shape=jax.ShapeDtypeStruct((srows, _LANE), jnp.float32),
    )(psum.reshape(_NC, srows, _LANE), pcnt.reshape(_NC, srows, _LANE))

    return mean2d.reshape(_SP)[:_NUM_SEGMENTS, None]
